# baseline (speedup 1.0000x reference)
import jax
import jax.numpy as jnp
from jax import lax
from jax.experimental import pallas as pl
from jax.experimental.pallas import tpu as pltpu
from jax.experimental.pallas import tpu_sc as plsc

VOCAB = 100000
EMBED_DIM = 64
BATCH = 4096
HIST = 50
HIST_PAD = 56
NUM_IDS = BATCH * HIST

NUM_WORKERS = 32
PER_WORKER = NUM_IDS // NUM_WORKERS
BATCHES_PER_WORKER = PER_WORKER // HIST
CB = 8
CHUNK = CB * HIST
NUM_CHUNKS = PER_WORKER // CHUNK
PAD_DIM = 128


def _sc_gather(table_padded, flat_ids):
    mesh = plsc.VectorSubcoreMesh(core_axis_name="c", subcore_axis_name="s")

    @pl.kernel(
        out_type=jax.ShapeDtypeStruct((BATCH, HIST_PAD, PAD_DIM), jnp.float32),
        mesh=mesh,
        scratch_types=[
            pltpu.VMEM((CHUNK,), jnp.int32),
            pltpu.VMEM((CHUNK,), jnp.int32),
            pltpu.VMEM((CHUNK + HIST_PAD - HIST, PAD_DIM), jnp.float32),
            pltpu.VMEM((CHUNK + HIST_PAD - HIST, PAD_DIM), jnp.float32),
            pltpu.SemaphoreType.DMA,
            pltpu.SemaphoreType.DMA,
            pltpu.SemaphoreType.DMA,
            pltpu.SemaphoreType.DMA,
            pltpu.SemaphoreType.DMA,
            pltpu.SemaphoreType.DMA,
        ],
    )
    def k(table_hbm, ids_hbm, out_hbm,
          idx0, idx1, rows0, rows1,
          isem0, isem1, gsem0, gsem1, ssem0, ssem1):
        wid = lax.axis_index("s") * 2 + lax.axis_index("c")
        base = wid * PER_WORKER
        bbase = wid * BATCHES_PER_WORKER

        idx_bufs = [idx0, idx1]
        row_bufs = [rows0, rows1]
        isems = [isem0, isem1]
        gsems = [gsem0, gsem1]
        ssems = [ssem0, ssem1]

        def idx_load(c, buf, sem):
            return pltpu.async_copy(
                ids_hbm.at[pl.ds(base + c * CHUNK, CHUNK)], buf, sem)

        def gather(idx_buf, row_buf, sem):
            return pltpu.async_copy(
                table_hbm.at[idx_buf], row_buf.at[pl.ds(0, CHUNK)], sem)

        def stores(c, row_buf, sem):
            hs = []
            for b in range(CB):
                hs.append(pltpu.async_copy(
                    row_buf.at[pl.ds(b * HIST, HIST_PAD)],
                    out_hbm.at[bbase + c * CB + b],
                    sem))
            return hs

        ih = [None, None]
        gh = [None, None]
        store_hs = [[], []]

        ih[0] = idx_load(0, idx_bufs[0], isems[0])
        ih[0].wait()
        gh[0] = gather(idx_bufs[0], row_bufs[0], gsems[0])
        if NUM_CHUNKS > 1:
            ih[1] = idx_load(1, idx_bufs[1], isems[1])

        for c in range(NUM_CHUNKS):
            cur = c & 1
            nxt = cur ^ 1
            gh[cur].wait()
            if c + 1 < NUM_CHUNKS:
                ih[nxt].wait()
                for h in store_hs[nxt]:
                    h.wait()
                store_hs[nxt] = []
                gh[nxt] = gather(idx_bufs[nxt], row_bufs[nxt], gsems[nxt])
                if c + 2 < NUM_CHUNKS:
                    ih[cur] = idx_load(c + 2, idx_bufs[cur], isems[cur])
            store_hs[cur] = stores(c, row_bufs[cur], ssems[cur])

        for hs in store_hs:
            for h in hs:
                h.wait()

    return k(table_padded, flat_ids)


PAD_BLK = 800
EPI_BATCHES = 8


def _tc_pad(table):
    def body(t_ref, o_ref):
        o_ref[:, :EMBED_DIM] = t_ref[...]

    return pl.pallas_call(
        body,
        grid=(VOCAB // PAD_BLK,),
        in_specs=[pl.BlockSpec((PAD_BLK, EMBED_DIM), lambda i: (i, 0))],
        out_specs=pl.BlockSpec((PAD_BLK, PAD_DIM), lambda i: (i, 0)),
        out_shape=jax.ShapeDtypeStruct((VOCAB, PAD_DIM), jnp.float32),
    )(table)


def _tc_epilogue(rows):
    def body(x_ref, o_ref):
        o_ref[...] = x_ref[:, :EMBED_DIM].reshape(EPI_BATCHES, HIST, EMBED_DIM)

    return pl.pallas_call(
        body,
        grid=(BATCH // EPI_BATCHES,),
        in_specs=[pl.BlockSpec((EPI_BATCHES * HIST, PAD_DIM),
                               lambda i: (i, 0))],
        out_specs=pl.BlockSpec((EPI_BATCHES, HIST, EMBED_DIM),
                               lambda i: (i, 0, 0)),
        out_shape=jax.ShapeDtypeStruct((BATCH, HIST, EMBED_DIM), jnp.float32),
    )(rows)


def kernel(vocab_ids, table):
    flat_ids = vocab_ids.astype(jnp.int32).reshape(NUM_IDS)
    table_padded = jnp.pad(table, ((0, 0), (0, PAD_DIM - EMBED_DIM)))
    out = _sc_gather(table_padded, flat_ids)
    return out[:, :HIST, :EMBED_DIM]


# speedup vs baseline: 1.0188x; 1.0188x over previous
import jax
import jax.numpy as jnp
from jax import lax
from jax.experimental import pallas as pl
from jax.experimental.pallas import tpu as pltpu
from jax.experimental.pallas import tpu_sc as plsc

VOCAB = 100000
EMBED_DIM = 64
BATCH = 4096
HIST = 50
HIST_PAD = 56
NUM_IDS = BATCH * HIST

NUM_WORKERS = 32
PER_WORKER = NUM_IDS // NUM_WORKERS
BATCHES_PER_WORKER = PER_WORKER // HIST
CB = 8
CHUNK = CB * HIST
NUM_CHUNKS = PER_WORKER // CHUNK
PAD_DIM = 128


def _sc_gather(table_padded, flat_ids):
    mesh = plsc.VectorSubcoreMesh(core_axis_name="c", subcore_axis_name="s")

    @pl.kernel(
        out_type=jax.ShapeDtypeStruct((BATCH, HIST, PAD_DIM), jnp.float32),
        mesh=mesh,
        scratch_types=[
            pltpu.VMEM((CHUNK,), jnp.int32),
            pltpu.VMEM((CHUNK,), jnp.int32),
            pltpu.VMEM((CHUNK + HIST_PAD - HIST, PAD_DIM), jnp.float32),
            pltpu.VMEM((CHUNK + HIST_PAD - HIST, PAD_DIM), jnp.float32),
            pltpu.SemaphoreType.DMA,
            pltpu.SemaphoreType.DMA,
            pltpu.SemaphoreType.DMA,
            pltpu.SemaphoreType.DMA,
            pltpu.SemaphoreType.DMA,
            pltpu.SemaphoreType.DMA,
        ],
    )
    def k(table_hbm, ids_hbm, out_hbm,
          idx0, idx1, rows0, rows1,
          isem0, isem1, gsem0, gsem1, ssem0, ssem1):
        wid = lax.axis_index("s") * 2 + lax.axis_index("c")
        base = wid * PER_WORKER
        bbase = wid * BATCHES_PER_WORKER

        idx_bufs = [idx0, idx1]
        row_bufs = [rows0, rows1]
        isems = [isem0, isem1]
        gsems = [gsem0, gsem1]
        ssems = [ssem0, ssem1]

        def idx_load(c, buf, sem):
            return pltpu.async_copy(
                ids_hbm.at[pl.ds(base + c * CHUNK, CHUNK)], buf, sem)

        def gather(idx_buf, row_buf, sem):
            return pltpu.async_copy(
                table_hbm.at[idx_buf], row_buf.at[pl.ds(0, CHUNK)], sem)

        def stores(c, row_buf, sem):
            hs = []
            for b in range(CB):
                hs.append(pltpu.async_copy(
                    row_buf.at[pl.ds(b * HIST, HIST)],
                    out_hbm.at[bbase + c * CB + b],
                    sem))
            return hs

        ih = [None, None]
        gh = [None, None]
        store_hs = [[], []]

        ih[0] = idx_load(0, idx_bufs[0], isems[0])
        ih[0].wait()
        gh[0] = gather(idx_bufs[0], row_bufs[0], gsems[0])
        if NUM_CHUNKS > 1:
            ih[1] = idx_load(1, idx_bufs[1], isems[1])

        for c in range(NUM_CHUNKS):
            cur = c & 1
            nxt = cur ^ 1
            gh[cur].wait()
            if c + 1 < NUM_CHUNKS:
                ih[nxt].wait()
                for h in store_hs[nxt]:
                    h.wait()
                store_hs[nxt] = []
                gh[nxt] = gather(idx_bufs[nxt], row_bufs[nxt], gsems[nxt])
                if c + 2 < NUM_CHUNKS:
                    ih[cur] = idx_load(c + 2, idx_bufs[cur], isems[cur])
            store_hs[cur] = stores(c, row_bufs[cur], ssems[cur])

        for hs in store_hs:
            for h in hs:
                h.wait()

    return k(table_padded, flat_ids)


PAD_BLK = 800
EPI_BATCHES = 8


def _tc_pad(table):
    def body(t_ref, o_ref):
        o_ref[:, :EMBED_DIM] = t_ref[...]

    return pl.pallas_call(
        body,
        grid=(VOCAB // PAD_BLK,),
        in_specs=[pl.BlockSpec((PAD_BLK, EMBED_DIM), lambda i: (i, 0))],
        out_specs=pl.BlockSpec((PAD_BLK, PAD_DIM), lambda i: (i, 0)),
        out_shape=jax.ShapeDtypeStruct((VOCAB, PAD_DIM), jnp.float32),
    )(table)


def _tc_epilogue(rows):
    def body(x_ref, o_ref):
        o_ref[...] = x_ref[:, :EMBED_DIM].reshape(EPI_BATCHES, HIST, EMBED_DIM)

    return pl.pallas_call(
        body,
        grid=(BATCH // EPI_BATCHES,),
        in_specs=[pl.BlockSpec((EPI_BATCHES * HIST, PAD_DIM),
                               lambda i: (i, 0))],
        out_specs=pl.BlockSpec((EPI_BATCHES, HIST, EMBED_DIM),
                               lambda i: (i, 0, 0)),
        out_shape=jax.ShapeDtypeStruct((BATCH, HIST, EMBED_DIM), jnp.float32),
    )(rows)


def kernel(vocab_ids, table):
    flat_ids = vocab_ids.astype(jnp.int32).reshape(NUM_IDS)
    table_padded = jnp.concatenate(
        [table, jnp.zeros((VOCAB, PAD_DIM - EMBED_DIM), jnp.float32)], axis=1)
    out = _sc_gather(table_padded, flat_ids)
    return out[:, :, :EMBED_DIM]


# speedup vs baseline: 1.0194x; 1.0006x over previous
import jax
import jax.numpy as jnp
from jax import lax
from jax.experimental import pallas as pl
from jax.experimental.pallas import tpu as pltpu
from jax.experimental.pallas import tpu_sc as plsc

VOCAB = 100000
EMBED_DIM = 64
BATCH = 4096
HIST = 50
HIST_PAD = 56
NUM_IDS = BATCH * HIST

NUM_WORKERS = 32
PER_WORKER = NUM_IDS // NUM_WORKERS
BATCHES_PER_WORKER = PER_WORKER // HIST
CB = 8
CHUNK = CB * HIST
NUM_CHUNKS = PER_WORKER // CHUNK
PAD_DIM = 128


def _sc_gather(table_padded, flat_ids):
    mesh = plsc.VectorSubcoreMesh(core_axis_name="c", subcore_axis_name="s")

    @pl.kernel(
        out_type=jax.ShapeDtypeStruct((BATCH, HIST, PAD_DIM), jnp.float32),
        mesh=mesh,
        scratch_types=[
            pltpu.VMEM((CHUNK,), jnp.int32),
            pltpu.VMEM((CHUNK,), jnp.int32),
            pltpu.VMEM((CHUNK + HIST_PAD - HIST, PAD_DIM), jnp.float32),
            pltpu.VMEM((CHUNK + HIST_PAD - HIST, PAD_DIM), jnp.float32),
            pltpu.SemaphoreType.DMA,
            pltpu.SemaphoreType.DMA,
            pltpu.SemaphoreType.DMA,
            pltpu.SemaphoreType.DMA,
            pltpu.SemaphoreType.DMA,
            pltpu.SemaphoreType.DMA,
        ],
    )
    def k(table_hbm, ids_hbm, out_hbm,
          idx0, idx1, rows0, rows1,
          isem0, isem1, gsem0, gsem1, ssem0, ssem1):
        wid = lax.axis_index("s") * 2 + lax.axis_index("c")
        base = wid * PER_WORKER
        bbase = wid * BATCHES_PER_WORKER

        idx_bufs = [idx0, idx1]
        row_bufs = [rows0, rows1]
        isems = [isem0, isem1]
        gsems = [gsem0, gsem1]
        ssems = [ssem0, ssem1]

        def idx_load(c, buf, sem):
            return pltpu.async_copy(
                ids_hbm.at[pl.ds(base + c * CHUNK, CHUNK)], buf, sem)

        def gather(idx_buf, row_buf, sem):
            return pltpu.async_copy(
                table_hbm.at[idx_buf], row_buf.at[pl.ds(0, CHUNK)], sem)

        def stores(c, row_buf, sem):
            hs = []
            for b in range(CB):
                hs.append(pltpu.async_copy(
                    row_buf.at[pl.ds(b * HIST, HIST)],
                    out_hbm.at[bbase + c * CB + b],
                    sem))
            return hs

        ih = [None, None]
        gh = [None, None]
        store_hs = [[], []]

        ih[0] = idx_load(0, idx_bufs[0], isems[0])
        ih[0].wait()
        gh[0] = gather(idx_bufs[0], row_bufs[0], gsems[0])
        if NUM_CHUNKS > 1:
            ih[1] = idx_load(1, idx_bufs[1], isems[1])

        for c in range(NUM_CHUNKS):
            cur = c & 1
            nxt = cur ^ 1
            gh[cur].wait()
            if c + 1 < NUM_CHUNKS:
                ih[nxt].wait()
                for h in store_hs[nxt]:
                    h.wait()
                store_hs[nxt] = []
                gh[nxt] = gather(idx_bufs[nxt], row_bufs[nxt], gsems[nxt])
                if c + 2 < NUM_CHUNKS:
                    ih[cur] = idx_load(c + 2, idx_bufs[cur], isems[cur])
            store_hs[cur] = stores(c, row_bufs[cur], ssems[cur])

        for hs in store_hs:
            for h in hs:
                h.wait()

    return k(table_padded, flat_ids)


PAD_BLK = 800
EPI_BATCHES = 8


def _tc_pad(table):
    def body(t_ref, o_ref):
        o_ref[:, :EMBED_DIM] = t_ref[...]

    return pl.pallas_call(
        body,
        grid=(VOCAB // PAD_BLK,),
        in_specs=[pl.BlockSpec((PAD_BLK, EMBED_DIM), lambda i: (i, 0))],
        out_specs=pl.BlockSpec((PAD_BLK, PAD_DIM), lambda i: (i, 0)),
        out_shape=jax.ShapeDtypeStruct((VOCAB, PAD_DIM), jnp.float32),
    )(table)


def _tc_epilogue(rows):
    def body(x_ref, o_ref):
        o_ref[...] = x_ref[:, :EMBED_DIM].reshape(EPI_BATCHES, HIST, EMBED_DIM)

    return pl.pallas_call(
        body,
        grid=(BATCH // EPI_BATCHES,),
        in_specs=[pl.BlockSpec((EPI_BATCHES * HIST, PAD_DIM),
                               lambda i: (i, 0))],
        out_specs=pl.BlockSpec((EPI_BATCHES, HIST, EMBED_DIM),
                               lambda i: (i, 0, 0)),
        out_shape=jax.ShapeDtypeStruct((BATCH, HIST, EMBED_DIM), jnp.float32),
    )(rows)


def kernel(vocab_ids, table):
    flat_ids = vocab_ids.astype(jnp.int32).reshape(NUM_IDS)
    table_padded = jax.lax.dynamic_update_slice(
        jnp.zeros((VOCAB, PAD_DIM), jnp.float32), table, (0, 0))
    out = _sc_gather(table_padded, flat_ids)
    return out[:, :, :EMBED_DIM]


# speedup vs baseline: 1.0207x; 1.0012x over previous
import jax
import jax.numpy as jnp
from jax import lax
from jax.experimental import pallas as pl
from jax.experimental.pallas import tpu as pltpu
from jax.experimental.pallas import tpu_sc as plsc

VOCAB = 100000
EMBED_DIM = 64
BATCH = 4096
HIST = 50
HIST_PAD = 56
NUM_IDS = BATCH * HIST

NUM_WORKERS = 32
PER_WORKER = NUM_IDS // NUM_WORKERS
BATCHES_PER_WORKER = PER_WORKER // HIST
CB = 8
CHUNK = CB * HIST
NUM_CHUNKS = PER_WORKER // CHUNK
PAD_DIM = 128


def _sc_gather(table_padded, flat_ids):
    mesh = plsc.VectorSubcoreMesh(core_axis_name="c", subcore_axis_name="s")

    @pl.kernel(
        out_type=jax.ShapeDtypeStruct((BATCH, HIST, PAD_DIM), jnp.float32),
        mesh=mesh,
        scratch_types=[
            pltpu.VMEM((CHUNK,), jnp.int32),
            pltpu.VMEM((CHUNK,), jnp.int32),
            pltpu.VMEM((CHUNK + HIST_PAD - HIST, PAD_DIM), jnp.float32),
            pltpu.VMEM((CHUNK + HIST_PAD - HIST, PAD_DIM), jnp.float32),
            pltpu.SemaphoreType.DMA,
            pltpu.SemaphoreType.DMA,
            pltpu.SemaphoreType.DMA,
            pltpu.SemaphoreType.DMA,
            pltpu.SemaphoreType.DMA,
            pltpu.SemaphoreType.DMA,
        ],
    )
    def k(table_hbm, ids_hbm, out_hbm,
          idx0, idx1, rows0, rows1,
          isem0, isem1, gsem0, gsem1, ssem0, ssem1):
        wid = lax.axis_index("s") * 2 + lax.axis_index("c")
        base = wid * PER_WORKER
        bbase = wid * BATCHES_PER_WORKER

        idx_bufs = [idx0, idx1]
        row_bufs = [rows0, rows1]
        isems = [isem0, isem1]
        gsems = [gsem0, gsem1]
        ssems = [ssem0, ssem1]

        def idx_load(c, buf, sem):
            return pltpu.async_copy(
                ids_hbm.at[pl.ds(base + c * CHUNK, CHUNK)], buf, sem)

        def gather(idx_buf, row_buf, sem):
            return pltpu.async_copy(
                table_hbm.at[idx_buf], row_buf.at[pl.ds(0, CHUNK)], sem)

        def stores(c, row_buf, sem):
            hs = []
            for b in range(CB):
                hs.append(pltpu.async_copy(
                    row_buf.at[pl.ds(b * HIST, HIST)],
                    out_hbm.at[bbase + c * CB + b],
                    sem))
            return hs

        ih = [None, None]
        gh = [None, None]
        store_hs = [[], []]

        ih[0] = idx_load(0, idx_bufs[0], isems[0])
        ih[0].wait()
        gh[0] = gather(idx_bufs[0], row_bufs[0], gsems[0])
        if NUM_CHUNKS > 1:
            ih[1] = idx_load(1, idx_bufs[1], isems[1])

        for c in range(NUM_CHUNKS):
            cur = c & 1
            nxt = cur ^ 1
            gh[cur].wait()
            if c + 1 < NUM_CHUNKS:
                ih[nxt].wait()
                for h in store_hs[nxt]:
                    h.wait()
                store_hs[nxt] = []
                gh[nxt] = gather(idx_bufs[nxt], row_bufs[nxt], gsems[nxt])
                if c + 2 < NUM_CHUNKS:
                    ih[cur] = idx_load(c + 2, idx_bufs[cur], isems[cur])
            store_hs[cur] = stores(c, row_bufs[cur], ssems[cur])

        for hs in store_hs:
            for h in hs:
                h.wait()

    return k(table_padded, flat_ids)


PAD_BLK = 800
EPI_BATCHES = 8


def _tc_pad(table):
    def body(t_ref, o_ref):
        o_ref[:, :EMBED_DIM] = t_ref[...]

    return pl.pallas_call(
        body,
        grid=(VOCAB // PAD_BLK,),
        in_specs=[pl.BlockSpec((PAD_BLK, EMBED_DIM), lambda i: (i, 0))],
        out_specs=pl.BlockSpec((PAD_BLK, PAD_DIM), lambda i: (i, 0)),
        out_shape=jax.ShapeDtypeStruct((VOCAB, PAD_DIM), jnp.float32),
    )(table)


def _tc_epilogue(rows):
    def body(x_ref, o_ref):
        o_ref[...] = x_ref[:, :EMBED_DIM].reshape(EPI_BATCHES, HIST, EMBED_DIM)

    return pl.pallas_call(
        body,
        grid=(BATCH // EPI_BATCHES,),
        in_specs=[pl.BlockSpec((EPI_BATCHES * HIST, PAD_DIM),
                               lambda i: (i, 0))],
        out_specs=pl.BlockSpec((EPI_BATCHES, HIST, EMBED_DIM),
                               lambda i: (i, 0, 0)),
        out_shape=jax.ShapeDtypeStruct((BATCH, HIST, EMBED_DIM), jnp.float32),
    )(rows)


def kernel(vocab_ids, table):
    flat_ids = vocab_ids.astype(jnp.int32).reshape(NUM_IDS)
    table_padded = jnp.pad(
        table.T, ((0, PAD_DIM - EMBED_DIM), (0, 0))).T
    out = _sc_gather(table_padded, flat_ids)
    return out[:, :, :EMBED_DIM]
